# baseline (speedup 1.0000x reference)
import functools

import jax
import jax.numpy as jnp
from jax import lax
from jax.experimental import pallas as pl
from jax.experimental.pallas import tpu as pltpu
from jax.experimental.pallas import tpu_sc as plsc

N_VERTS = 35709
ROW = N_VERTS * 3
BATCH = 128
TOTAL = BATCH * ROW
NVECS = TOTAL // 16
NUM_WORKERS = 32
ROWS_PER_W = BATCH // NUM_WORKERS

REST, EYE, MOUTH = 0, 1, 2
N_MOUTH = 1700 * 3 * BATCH
N_EYE = 1600 * 3 * BATCH
N_REST = (N_VERTS - 3300) * 3 * BATCH

CHV = 1536
LAST_CLV = 551
UNROLL = 8

_SEGS = ((0, 12000, REST), (12000, 14400, EYE), (14400, 24000, REST),
         (24000, 26400, EYE), (26400, 38400, REST), (38400, 43500, MOUTH),
         (43500, ROW, REST))

_CHUNKS = ((0, CHV), (CHV, CHV), (2 * CHV, CHV), (3 * CHV, CHV),
           (4 * CHV, LAST_CLV))

def _candidates(k):
    cs, cl = _CHUNKS[k]
    out = []
    for a, b, r in _SEGS:
        if a < (cs + cl) * 16 and b > cs * 16 - 15:
            out.append((a, b, r))
    return tuple(out)

_CAND = tuple(_candidates(k) for k in range(len(_CHUNKS)))


def _masked_add(xb, yb, v, lane, lo, hi, acc):
    d = jnp.abs(xb[v, :] - yb[v, :])
    msk = (lane >= lo) & (lane < hi)
    return acc + jnp.where(msk, d, 0.0)


def _bulk_sum(xb, yb, vlo, nvec):
    zero = jnp.zeros((16,), jnp.float32)
    n8 = nvec // UNROLL

    def body8(i, accs):
        b0, b1 = accs
        base = vlo + i * UNROLL
        for u in range(UNROLL):
            v = jnp.abs(xb[base + u, :] - yb[base + u, :])
            if u % 2 == 0:
                b0 = b0 + v
            else:
                b1 = b1 + v
        return (b0, b1)

    a0, a1 = lax.fori_loop(0, n8, body8, (zero, zero))
    rem_base = vlo + n8 * UNROLL

    def body1(i, a):
        return a + jnp.abs(xb[rem_base + i, :] - yb[rem_base + i, :])

    a0 = lax.fori_loop(0, nvec - n8 * UNROLL, body1, a0)
    return a0 + a1


def _span(xb, yb, lane, wstart, clv, a, b, acc):
    cl = clv * 16
    ja = jnp.clip(a - wstart, 0, cl)
    jb = jnp.clip(b - wstart, 0, cl)
    vlo = (ja + 15) >> 4
    vhi = jb >> 4
    hb = ja >> 4
    acc = _masked_add(xb, yb, hb, lane, ja - hb * 16,
                      jnp.minimum(jb, vlo * 16) - hb * 16, acc)
    nvec = jnp.maximum(vhi - vlo, 0)
    acc = acc + _bulk_sum(xb, yb, vlo, nvec)
    tlo = jnp.maximum(jnp.maximum(ja, vlo * 16), vhi * 16)
    acc = _masked_add(xb, yb, vhi, lane, tlo - vhi * 16, jb - vhi * 16, acc)
    return acc


def _region_l1_sc(x_hbm, y_hbm, out_hbm, xbuf0, xbuf1, ybuf0, ybuf1,
                  xt0, xt1, yt0, yt1, accbuf, sem0, sem1, semt):
    wid = lax.axis_index("s") * 2 + lax.axis_index("c")
    zero = jnp.zeros((16,), jnp.float32)
    lane = lax.iota(jnp.int32, 16)
    sems = (sem0, sem1)
    xbufs = (xbuf0, xbuf1)
    ybufs = (ybuf0, ybuf1)
    nchunks = len(_CHUNKS)

    def issue(avbase, c, slot):
        cs, cl = _CHUNKS[c]
        start = avbase + cs
        hx = pltpu.async_copy(x_hbm.at[pl.ds(start, cl)],
                              xbufs[slot].at[pl.ds(0, cl)], sems[slot])
        hy = pltpu.async_copy(y_hbm.at[pl.ds(start, cl)],
                              ybufs[slot].at[pl.ds(0, cl)], sems[slot])
        return hx, hy

    def row_body(r, accs):
        row = wid * ROWS_PER_W + r
        frow = row * ROW
        m = frow & 15
        avbase = (frow - m) >> 4
        acc = list(accs)

        t0v = avbase + 4 * CHV + LAST_CLV
        t1v = jnp.minimum(t0v + 1, NVECS - 1)
        ht = [pltpu.async_copy(x_hbm.at[pl.ds(t0v, 1)], xt0, semt),
              pltpu.async_copy(y_hbm.at[pl.ds(t0v, 1)], yt0, semt),
              pltpu.async_copy(x_hbm.at[pl.ds(t1v, 1)], xt1, semt),
              pltpu.async_copy(y_hbm.at[pl.ds(t1v, 1)], yt1, semt)]

        handles = [None, None]
        handles[0] = issue(avbase, 0, 0)
        handles[1] = issue(avbase, 1, 1)
        for c in range(nchunks):
            cs, cl = _CHUNKS[c]
            slot = c % 2
            hx, hy = handles[slot]
            hx.wait()
            hy.wait()
            xb, yb = xbufs[slot], ybufs[slot]
            wstart = cs * 16 - m
            for a, b, reg in _CAND[c]:
                acc[reg] = _span(xb, yb, lane, wstart, cl, a, b, acc[reg])
            if c + 2 < nchunks:
                handles[slot] = issue(avbase, c + 2, slot)

        for h in ht:
            h.wait()
        acc[REST] = _masked_add(xt0, yt0, 0, lane, 0,
                                jnp.minimum(m + 7, 16), acc[REST])
        acc[REST] = _masked_add(xt1, yt1, 0, lane, 0, m - 9, acc[REST])
        return tuple(acc)

    acc_rest, acc_eye, acc_mouth = lax.fori_loop(
        0, ROWS_PER_W, row_body, (zero, zero, zero))
    accbuf[0, :] = acc_rest
    accbuf[1, :] = acc_eye
    accbuf[2, :] = acc_mouth
    pltpu.sync_copy(accbuf, out_hbm.at[wid])


@functools.cache
def _build_sc_kernel():
    mesh = plsc.VectorSubcoreMesh(core_axis_name="c", subcore_axis_name="s")
    return functools.partial(
        pl.kernel,
        mesh=mesh,
        out_type=jax.ShapeDtypeStruct((NUM_WORKERS, 3, 16), jnp.float32),
        scratch_types=[
            pltpu.VMEM((CHV + 1, 16), jnp.float32),
            pltpu.VMEM((CHV + 1, 16), jnp.float32),
            pltpu.VMEM((CHV + 1, 16), jnp.float32),
            pltpu.VMEM((CHV + 1, 16), jnp.float32),
            pltpu.VMEM((1, 16), jnp.float32),
            pltpu.VMEM((1, 16), jnp.float32),
            pltpu.VMEM((1, 16), jnp.float32),
            pltpu.VMEM((1, 16), jnp.float32),
            pltpu.VMEM((3, 16), jnp.float32),
            pltpu.SemaphoreType.DMA,
            pltpu.SemaphoreType.DMA,
            pltpu.SemaphoreType.DMA,
        ],
        compiler_params=pltpu.CompilerParams(use_tc_tiling_on_sc=False),
    )(_region_l1_sc)


def kernel(input, target):
    x = input.reshape(NVECS, 16)
    y = target.reshape(NVECS, 16)
    partials = _build_sc_kernel()(x, y)
    sums = partials.sum(axis=(0, 2))
    mouth_loss = sums[MOUTH] / N_MOUTH
    eye_loss = sums[EYE] / N_EYE
    rest_loss = sums[REST] / N_REST
    return (mouth_loss, eye_loss, rest_loss)


# speedup vs baseline: 1.0013x; 1.0013x over previous
import functools

import jax
import jax.numpy as jnp
from jax import lax
from jax.experimental import pallas as pl
from jax.experimental.pallas import tpu as pltpu
from jax.experimental.pallas import tpu_sc as plsc

N_VERTS = 35709
ROW = N_VERTS * 3
BATCH = 128
TOTAL = BATCH * ROW
NVECS = TOTAL // 16
NUM_WORKERS = 32
ROWS_PER_W = BATCH // NUM_WORKERS

REST, EYE, MOUTH = 0, 1, 2
N_MOUTH = 1700 * 3 * BATCH
N_EYE = 1600 * 3 * BATCH
N_REST = (N_VERTS - 3300) * 3 * BATCH

CHV = 1536
LAST_CLV = 551
UNROLL = 8

_SEGS = ((0, 12000, REST), (12000, 14400, EYE), (14400, 24000, REST),
         (24000, 26400, EYE), (26400, 38400, REST), (38400, 43500, MOUTH),
         (43500, ROW, REST))

_CHUNKS = ((0, CHV), (CHV, CHV), (2 * CHV, CHV), (3 * CHV, CHV),
           (4 * CHV, LAST_CLV))


def _masked_add(xb, yb, v, lane, lo, hi, acc):
    d = jnp.abs(xb[v, :] - yb[v, :])
    msk = (lane >= lo) & (lane < hi)
    return acc + jnp.where(msk, d, 0.0)


def _bulk_sum(xb, yb, vlo, nvec):
    zero = jnp.zeros((16,), jnp.float32)
    n8 = nvec // UNROLL

    def body8(i, accs):
        b0, b1 = accs
        base = vlo + i * UNROLL
        for u in range(UNROLL):
            v = jnp.abs(xb[base + u, :] - yb[base + u, :])
            if u % 2 == 0:
                b0 = b0 + v
            else:
                b1 = b1 + v
        return (b0, b1)

    a0, a1 = lax.fori_loop(0, n8, body8, (zero, zero))
    rem_base = vlo + n8 * UNROLL
    for i in range(nvec - n8 * UNROLL):
        a1 = a1 + jnp.abs(xb[rem_base + i, :] - yb[rem_base + i, :])
    return a0 + a1


def _span_plan(clv, arel, brel):
    bs = max(-(-(arel + 15) // 16), 0)
    be = max(min((brel - 16) // 16 + 1, clv), 0)
    be = max(be, bs)
    h0 = max(arel // 16, 0)
    t1 = min(-(-(brel + 15) // 16), clv)
    edges = list(range(h0, min(bs, clv))) + list(range(max(be, h0), t1))
    return tuple(edges), bs, be - bs


def _candidates(k):
    cs, cl = _CHUNKS[k]
    out = []
    for a, b, r in _SEGS:
        if a < (cs + cl) * 16 and b > cs * 16 - 15:
            arel, brel = a - 16 * cs, b - 16 * cs
            out.append((arel, brel, r, _span_plan(cl, arel, brel)))
    return tuple(out)

_CAND = tuple(_candidates(k) for k in range(len(_CHUNKS)))


def _span_sum(xb, yb, lane, m, plan, arel, brel, acc):
    edges, bs, nv = plan
    for v in edges:
        acc = _masked_add(xb, yb, v, lane, arel - 16 * v + m,
                          brel - 16 * v + m, acc)
    if nv > 0:
        acc = acc + _bulk_sum(xb, yb, bs, nv)
    return acc


def _region_l1_sc(x_hbm, y_hbm, out_hbm, xbuf0, xbuf1, ybuf0, ybuf1,
                  xt0, xt1, yt0, yt1, accbuf, sem0, sem1, semt):
    wid = lax.axis_index("s") * 2 + lax.axis_index("c")
    zero = jnp.zeros((16,), jnp.float32)
    lane = lax.iota(jnp.int32, 16)
    sems = (sem0, sem1)
    xbufs = (xbuf0, xbuf1)
    ybufs = (ybuf0, ybuf1)
    nchunks = len(_CHUNKS)

    def issue(avbase, c, slot):
        cs, cl = _CHUNKS[c]
        start = avbase + cs
        hx = pltpu.async_copy(x_hbm.at[pl.ds(start, cl)],
                              xbufs[slot].at[pl.ds(0, cl)], sems[slot])
        hy = pltpu.async_copy(y_hbm.at[pl.ds(start, cl)],
                              ybufs[slot].at[pl.ds(0, cl)], sems[slot])
        return hx, hy

    def row_body(r, accs):
        row = wid * ROWS_PER_W + r
        frow = row * ROW
        m = frow & 15
        avbase = (frow - m) >> 4
        acc = list(accs)

        t0v = avbase + 4 * CHV + LAST_CLV
        t1v = jnp.minimum(t0v + 1, NVECS - 1)
        ht = [pltpu.async_copy(x_hbm.at[pl.ds(t0v, 1)], xt0, semt),
              pltpu.async_copy(y_hbm.at[pl.ds(t0v, 1)], yt0, semt),
              pltpu.async_copy(x_hbm.at[pl.ds(t1v, 1)], xt1, semt),
              pltpu.async_copy(y_hbm.at[pl.ds(t1v, 1)], yt1, semt)]

        handles = [None, None]
        handles[0] = issue(avbase, 0, 0)
        handles[1] = issue(avbase, 1, 1)
        for c in range(nchunks):
            cs, cl = _CHUNKS[c]
            slot = c % 2
            hx, hy = handles[slot]
            hx.wait()
            hy.wait()
            xb, yb = xbufs[slot], ybufs[slot]
            for arel, brel, reg, plan in _CAND[c]:
                acc[reg] = _span_sum(xb, yb, lane, m, plan, arel, brel,
                                     acc[reg])
            if c + 2 < nchunks:
                handles[slot] = issue(avbase, c + 2, slot)

        for h in ht:
            h.wait()
        acc[REST] = _masked_add(xt0, yt0, 0, lane, 0,
                                jnp.minimum(m + 7, 16), acc[REST])
        acc[REST] = _masked_add(xt1, yt1, 0, lane, 0, m - 9, acc[REST])
        return tuple(acc)

    acc_rest, acc_eye, acc_mouth = lax.fori_loop(
        0, ROWS_PER_W, row_body, (zero, zero, zero))
    accbuf[0, :] = acc_rest
    accbuf[1, :] = acc_eye
    accbuf[2, :] = acc_mouth
    pltpu.sync_copy(accbuf, out_hbm.at[wid])


@functools.cache
def _build_sc_kernel():
    mesh = plsc.VectorSubcoreMesh(core_axis_name="c", subcore_axis_name="s")
    return functools.partial(
        pl.kernel,
        mesh=mesh,
        out_type=jax.ShapeDtypeStruct((NUM_WORKERS, 3, 16), jnp.float32),
        scratch_types=[
            pltpu.VMEM((CHV + 1, 16), jnp.float32),
            pltpu.VMEM((CHV + 1, 16), jnp.float32),
            pltpu.VMEM((CHV + 1, 16), jnp.float32),
            pltpu.VMEM((CHV + 1, 16), jnp.float32),
            pltpu.VMEM((1, 16), jnp.float32),
            pltpu.VMEM((1, 16), jnp.float32),
            pltpu.VMEM((1, 16), jnp.float32),
            pltpu.VMEM((1, 16), jnp.float32),
            pltpu.VMEM((3, 16), jnp.float32),
            pltpu.SemaphoreType.DMA,
            pltpu.SemaphoreType.DMA,
            pltpu.SemaphoreType.DMA,
        ],
        compiler_params=pltpu.CompilerParams(use_tc_tiling_on_sc=False),
    )(_region_l1_sc)


def kernel(input, target):
    x = input.reshape(NVECS, 16)
    y = target.reshape(NVECS, 16)
    partials = _build_sc_kernel()(x, y)
    sums = partials.sum(axis=(0, 2))
    mouth_loss = sums[MOUTH] / N_MOUTH
    eye_loss = sums[EYE] / N_EYE
    rest_loss = sums[REST] / N_REST
    return (mouth_loss, eye_loss, rest_loss)


# speedup vs baseline: 62.2203x; 62.1379x over previous
import functools

import jax
import jax.numpy as jnp
from jax import lax
from jax.experimental import pallas as pl
from jax.experimental.pallas import tpu as pltpu
from jax.experimental.pallas import tpu_sc as plsc

N_VERTS = 35709
ROW = N_VERTS * 3
ROW_PAD = 107136
BATCH = 128
NUM_WORKERS = 32
ROWS_PER_W = BATCH // NUM_WORKERS

REST, EYE, MOUTH, SPLIT = 0, 1, 2, 3
N_MOUTH = 1700 * 3 * BATCH
N_EYE = 1600 * 3 * BATCH
N_REST = (N_VERTS - 3300) * 3 * BATCH

CH = 24576
UNROLL = 8

_SEGS = ((0, 12000, REST), (12000, 14400, EYE), (14400, 24000, REST),
         (24000, 26400, EYE), (26400, 38400, REST), (38400, 43488, MOUTH),
         (43488, 43504, SPLIT), (43504, ROW_PAD, REST))


def _chunk_table():
    chunks = []
    cs = 0
    while cs < ROW_PAD:
        ce = min(cs + CH, ROW_PAD)
        spans = []
        for s, e, kind in _SEGS:
            lo, hi = max(s, cs), min(e, ce)
            if lo < hi:
                spans.append((lo - cs, hi - lo, kind))
        chunks.append((cs, ce - cs, tuple(spans)))
        cs = ce
    return tuple(chunks)

DMA_CHUNKS = _chunk_table()


def _span_sum(xb, yb, off, nvec):
    a0 = jnp.zeros((16,), jnp.float32)
    a1 = jnp.zeros((16,), jnp.float32)
    n_u = nvec // UNROLL

    if n_u > 0:
        def body(i, accs):
            b0, b1 = accs
            base = off + i * (16 * UNROLL)
            for u in range(UNROLL):
                o = base + u * 16
                v = jnp.abs(xb[pl.ds(o, 16)] - yb[pl.ds(o, 16)])
                if u % 2 == 0:
                    b0 = b0 + v
                else:
                    b1 = b1 + v
            return (b0, b1)
        a0, a1 = lax.fori_loop(0, n_u, body, (a0, a1))
    base = off + n_u * (16 * UNROLL)
    for u in range(nvec % UNROLL):
        o = base + u * 16
        v = jnp.abs(xb[pl.ds(o, 16)] - yb[pl.ds(o, 16)])
        if u % 2 == 0:
            a0 = a0 + v
        else:
            a1 = a1 + v
    return a0 + a1


def _region_l1_sc(x_hbm, y_hbm, out_hbm, xbuf0, xbuf1, ybuf0, ybuf1,
                  accbuf, sem0, sem1):
    wid = lax.axis_index("s") * 2 + lax.axis_index("c")
    zero = jnp.zeros((16,), jnp.float32)
    lane = lax.iota(jnp.int32, 16)
    sems = (sem0, sem1)
    xbufs = (xbuf0, xbuf1)
    ybufs = (ybuf0, ybuf1)
    nchunks = len(DMA_CHUNKS)

    def issue(row, c, slot):
        cs, cl, _ = DMA_CHUNKS[c]
        hx = pltpu.async_copy(x_hbm.at[row, pl.ds(cs, cl)],
                              xbufs[slot].at[pl.ds(0, cl)], sems[slot])
        hy = pltpu.async_copy(y_hbm.at[row, pl.ds(cs, cl)],
                              ybufs[slot].at[pl.ds(0, cl)], sems[slot])
        return hx, hy

    def row_body(r, accs):
        row = wid * ROWS_PER_W + r
        acc = list(accs)
        handles = [None, None]
        handles[0] = issue(row, 0, 0)
        handles[1] = issue(row, 1, 1)
        for c, (cs, cl, spans) in enumerate(DMA_CHUNKS):
            slot = c % 2
            hx, hy = handles[slot]
            hx.wait()
            hy.wait()
            xb, yb = xbufs[slot], ybufs[slot]
            for off, ln, kind in spans:
                if kind == SPLIT:
                    d = jnp.abs(xb[pl.ds(off, 16)] - yb[pl.ds(off, 16)])
                    acc[MOUTH] = acc[MOUTH] + jnp.where(lane < 12, d, 0.0)
                    acc[REST] = acc[REST] + jnp.where(lane >= 12, d, 0.0)
                else:
                    acc[kind] = acc[kind] + _span_sum(xb, yb, off, ln // 16)
            if c + 2 < nchunks:
                handles[slot] = issue(row, c + 2, slot)
        return tuple(acc)

    acc_rest, acc_eye, acc_mouth = lax.fori_loop(
        0, ROWS_PER_W, row_body, (zero, zero, zero))
    accbuf[0, :] = acc_rest
    accbuf[1, :] = acc_eye
    accbuf[2, :] = acc_mouth
    pltpu.sync_copy(accbuf, out_hbm.at[wid])


@functools.cache
def _build_sc_kernel():
    mesh = plsc.VectorSubcoreMesh(core_axis_name="c", subcore_axis_name="s")
    return functools.partial(
        pl.kernel,
        mesh=mesh,
        out_type=jax.ShapeDtypeStruct((NUM_WORKERS, 3, 16), jnp.float32),
        scratch_types=[
            pltpu.VMEM((CH,), jnp.float32),
            pltpu.VMEM((CH,), jnp.float32),
            pltpu.VMEM((CH,), jnp.float32),
            pltpu.VMEM((CH,), jnp.float32),
            pltpu.VMEM((3, 16), jnp.float32),
            pltpu.SemaphoreType.DMA,
            pltpu.SemaphoreType.DMA,
        ],
        compiler_params=pltpu.CompilerParams(use_tc_tiling_on_sc=False),
    )(_region_l1_sc)


def kernel(input, target):
    x = input.reshape(BATCH, ROW)
    y = target.reshape(BATCH, ROW)
    pad = ROW_PAD - ROW
    x = jnp.pad(x, ((0, 0), (0, pad)))
    y = jnp.pad(y, ((0, 0), (0, pad)))
    partials = _build_sc_kernel()(x, y)
    sums = partials.sum(axis=(0, 2))
    mouth_loss = sums[MOUTH] / N_MOUTH
    eye_loss = sums[EYE] / N_EYE
    rest_loss = sums[REST] / N_REST
    return (mouth_loss, eye_loss, rest_loss)
